# baseline (speedup 1.0000x reference)
import jax
import jax.numpy as jnp
from jax import lax
from jax.experimental import pallas as pl
from jax.experimental.pallas import tpu as pltpu
from jax.experimental.pallas import tpu_sc as plsc

NFEAT = 1000000
NEMB = 64
B = 16384
F = 26

NC = 2
NS = 16
NW = NC * NS

BF = B * F
N_PER_W = BF // NW
G = 128
NG = N_PER_W // G
NBUF = 4
NSTEP = NG // NBUF
CB = B // G


def _emb_body(table_hbm, idx_hbm, val_hbm, out_hbm,
              idx_v, val_v, rows_v, obuf_v, gsems, osems):
    wid = lax.axis_index("s") * NC + lax.axis_index("c")
    base = wid * N_PER_W
    g0 = wid * NG

    pltpu.sync_copy(idx_hbm.at[pl.ds(base, N_PER_W)], idx_v)
    pltpu.sync_copy(val_hbm.at[pl.ds(base, N_PER_W)], val_v)

    iota16 = lax.iota(jnp.int32, 16)

    def fire_gather(g, b):
        pltpu.make_async_copy(
            table_hbm.at[idx_v.at[pl.ds(g * G, G)]],
            rows_v.at[b],
            gsems.at[b],
        ).start()

    def wait_gather(b):
        pltpu.make_async_copy(
            table_hbm.at[idx_v.at[pl.ds(0, G)]],
            rows_v.at[b],
            gsems.at[b],
        ).wait()

    def start_store(g, b):
        gg = g0 + g
        f = gg >> 7
        c = gg & (CB - 1)
        pltpu.make_async_copy(
            obuf_v.at[b],
            out_hbm.at[f, :, c],
            osems.at[b],
        ).start()

    def wait_store(b):
        pltpu.make_async_copy(
            obuf_v.at[b],
            out_hbm.at[0, :, 0],
            osems.at[b],
        ).wait()

    for bi in range(NBUF):
        fire_gather(bi, bi)

    def step(t, _):
        for bi in range(NBUF):
            g = t * NBUF + bi
            wait_gather(bi)

            @pl.when(t > 0)
            def _():
                wait_store(bi)

            src = rows_v.at[bi]
            dst = obuf_v.at[bi]

            for lq in range(G // 16):
                vv = val_v[pl.ds(g * G + lq * 16, 16)]
                lrow = iota16 + (lq * 16)
                for r in range(8):
                    for s in range(8):
                        ecol = jnp.full((16,), r * 8 + s, jnp.int32)
                        col = plsc.load_gather(src, [lrow, ecol])
                        dst[r, s, pl.ds(lq * 16, 16)] = col * vv

            @pl.when(g + NBUF < NG)
            def _():
                fire_gather(g + NBUF, bi)

            start_store(g, bi)
        return 0

    lax.fori_loop(0, NSTEP, step, 0)

    for bi in range(NBUF):
        wait_store(bi)


def _make_emb():
    mesh = plsc.VectorSubcoreMesh(core_axis_name="c", subcore_axis_name="s")
    return pl.kernel(
        _emb_body,
        out_type=jax.ShapeDtypeStruct((F, NEMB // 8, CB, 8, G), jnp.float32),
        mesh=mesh,
        compiler_params=pltpu.CompilerParams(
            use_tc_tiling_on_sc=False, needs_layout_passes=False),
        scratch_types=[
            pltpu.VMEM((N_PER_W,), jnp.int32),
            pltpu.VMEM((N_PER_W,), jnp.float32),
            pltpu.VMEM((NBUF, G, NEMB), jnp.float32),
            pltpu.VMEM((NBUF, NEMB // 8, 8, G), jnp.float32),
            pltpu.SemaphoreType.DMA((NBUF,)),
            pltpu.SemaphoreType.DMA((NBUF,)),
        ],
    )


@jax.jit
def kernel(id, value, table):
    idx_f = id.T.reshape(BF)
    val_f = value.T.reshape(BF)
    out5 = _make_emb()(table, idx_f, val_f)
    return out5.transpose(2, 4, 0, 1, 3).reshape(B, F, NEMB)


# speedup vs baseline: 1.6523x; 1.6523x over previous
import jax
import jax.numpy as jnp
from jax import lax
from jax.experimental import pallas as pl
from jax.experimental.pallas import tpu as pltpu
from jax.experimental.pallas import tpu_sc as plsc

NFEAT = 1000000
NEMB = 64
B = 16384
F = 26

NC = 2
NS = 16
NW = NC * NS

BF = B * F
N_PER_W = BF // NW
G = 128
NG = N_PER_W // G
NBUF = 4
NSTEP = NG // NBUF
CB = B // G
OPITCH = G + 1


def _emb_body(table_hbm, idx_hbm, val_hbm, out_hbm,
              idx_v, val_v, rows_v, obuf_v, gsems, osems):
    wid = lax.axis_index("s") * NC + lax.axis_index("c")
    base = wid * N_PER_W
    g0 = wid * NG

    pltpu.sync_copy(idx_hbm.at[pl.ds(base, N_PER_W)], idx_v)
    pltpu.sync_copy(val_hbm.at[pl.ds(base, N_PER_W)], val_v)

    iota16 = lax.iota(jnp.int32, 16)

    def fire_gather(g, b):
        pltpu.make_async_copy(
            table_hbm.at[idx_v.at[pl.ds(g * G, G)]],
            rows_v.at[b],
            gsems.at[b],
        ).start()

    def wait_gather(b):
        pltpu.make_async_copy(
            table_hbm.at[idx_v.at[pl.ds(0, G)]],
            rows_v.at[b],
            gsems.at[b],
        ).wait()

    def start_store(g, b):
        gg = g0 + g
        f = gg >> 7
        c = gg & (CB - 1)
        pltpu.make_async_copy(
            obuf_v.at[b, :, :, pl.ds(0, G)],
            out_hbm.at[f, :, c],
            osems.at[b],
        ).start()

    def wait_store(b):
        pltpu.make_async_copy(
            obuf_v.at[b, :, :, pl.ds(0, G)],
            out_hbm.at[0, :, 0],
            osems.at[b],
        ).wait()

    e_r = [(16 * q + iota16) >> 3 for q in range(NEMB // 16)]
    e_s = [(16 * q + iota16) & 7 for q in range(NEMB // 16)]

    for bi in range(NBUF):
        fire_gather(bi, bi)

    def step(t, _):
        for bi in range(NBUF):
            g = t * NBUF + bi
            wait_gather(bi)

            @pl.when(t > 0)
            def _():
                wait_store(bi)

            src = rows_v.at[bi]
            dst = obuf_v.at[bi]

            def lq_body(lq, _):
                vv = val_v[pl.ds(g * G + lq * 16, 16)]
                for i in range(16):
                    lane = lq * 16 + i
                    v = vv[i]
                    lcol = jnp.full((16,), lane, jnp.int32)
                    for q in range(NEMB // 16):
                        row = src[lane, pl.ds(q * 16, 16)]
                        plsc.store_scatter(
                            dst, [e_r[q], e_s[q], lcol], row * v)
                return 0

            lax.fori_loop(0, G // 16, lq_body, 0)

            @pl.when(g + NBUF < NG)
            def _():
                fire_gather(g + NBUF, bi)

            start_store(g, bi)
        return 0

    lax.fori_loop(0, NSTEP, step, 0)

    for bi in range(NBUF):
        wait_store(bi)


def _make_emb():
    mesh = plsc.VectorSubcoreMesh(core_axis_name="c", subcore_axis_name="s")
    return pl.kernel(
        _emb_body,
        out_type=jax.ShapeDtypeStruct((F, NEMB // 8, CB, 8, G), jnp.float32),
        mesh=mesh,
        compiler_params=pltpu.CompilerParams(
            use_tc_tiling_on_sc=False, needs_layout_passes=False),
        scratch_types=[
            pltpu.VMEM((N_PER_W,), jnp.int32),
            pltpu.VMEM((N_PER_W,), jnp.float32),
            pltpu.VMEM((NBUF, G, NEMB), jnp.float32),
            pltpu.VMEM((NBUF, NEMB // 8, 8, OPITCH), jnp.float32),
            pltpu.SemaphoreType.DMA((NBUF,)),
            pltpu.SemaphoreType.DMA((NBUF,)),
        ],
    )


@jax.jit
def kernel(id, value, table):
    idx_f = id.T.reshape(BF)
    val_f = value.T.reshape(BF)
    out5 = _make_emb()(table, idx_f, val_f)
    return out5.transpose(2, 4, 0, 1, 3).reshape(B, F, NEMB)
